# Initial kernel scaffold; baseline (speedup 1.0000x reference)
import jax
import jax.numpy as jnp
from jax.experimental import pallas as pl


def kernel(x, edge_index, edge_attr, atom_table, bond_tables, W1, b1, g1, be1, W2, b2, bn_g, bn_b, eps_param):
    raise NotImplementedError("write your pallas kernel here")


# speedup vs baseline: 3.5526x; 3.5526x over previous
import functools

import numpy as np
import jax
import jax.numpy as jnp
from jax import lax
from jax.experimental import pallas as pl
from jax.experimental.pallas import tpu as pltpu
from jax.experimental.pallas import tpu_sc as plsc

_ATOM_DIMS = (119, 4, 12, 12, 10, 6, 6, 2, 2)
_BOND_DIMS = (5, 6, 2)
_N, _E, _D, _L = 10000, 320000, 128, 6
_AF = sum(_ATOM_DIMS)
_BF = sum(_BOND_DIMS)

_NC, _NS = 2, 16
_NW = _NC * _NS
_EPW = _E // _NW
_CH = 80
_NCHUNK = _EPW // _CH
_NPAD = 10240
_RPS = _NPAD // _NS

_oh = np.zeros((60, _BF), np.float32)
for _c in range(60):
    _a0, _r = divmod(_c, 12)
    _a1, _a2 = divmod(_r, 2)
    _oh[_c, _a0] = 1.0
    _oh[_c, 5 + _a1] = 1.0
    _oh[_c, 11 + _a2] = 1.0
_ONEHOT60 = _oh


def _first_argmax_onehot(v, d):
    m = jnp.max(v, axis=1, keepdims=True)
    iota = lax.broadcasted_iota(jnp.int32, v.shape, 1)
    idx = jnp.min(jnp.where(v == m, iota, d), axis=1, keepdims=True)
    return iota == idx, idx


def _h0_body(x_ref, tab_ref, out_ref):
    xv = x_ref[...]
    cols = []
    start = 0
    for d in _ATOM_DIMS:
        onehot, _ = _first_argmax_onehot(xv[:, start:start + d], d)
        cols.append(onehot.astype(jnp.float32))
        start += d
    onehot = jnp.concatenate(cols, axis=1)
    out_ref[...] = lax.dot(onehot, tab_ref[...],
                           precision=lax.Precision.HIGHEST)


def _embed_h0(x, atom_table):
    B = 2000
    return pl.pallas_call(
        _h0_body,
        grid=(_N // B,),
        in_specs=[pl.BlockSpec((B, _AF), lambda i: (i, 0)),
                  pl.BlockSpec((_AF, _D), lambda i: (0, 0))],
        out_specs=pl.BlockSpec((B, _D), lambda i: (i, 0)),
        out_shape=jax.ShapeDtypeStruct((_N, _D), jnp.float32),
    )(x, atom_table)


def _combo_body(ea_ref, out_ref, *, block):
    v = ea_ref[...]
    idxs = []
    start = 0
    for d in _BOND_DIMS:
        _, idx = _first_argmax_onehot(v[:, start:start + d], d)
        idxs.append(idx[:, 0])
        start += d
    i = pl.program_id(0)
    out_ref[pl.ds(i * block, block)] = idxs[0] * 12 + idxs[1] * 2 + idxs[2]


def _combo_idx(edge_attr):
    B = 6400
    return pl.pallas_call(
        functools.partial(_combo_body, block=B),
        grid=(_E // B,),
        in_specs=[pl.BlockSpec((B, _BF), lambda i: (i, 0))],
        out_specs=pl.BlockSpec((_E,), lambda i: (0,)),
        out_shape=jax.ShapeDtypeStruct((_E,), jnp.int32),
    )(edge_attr)


def _mlp_body(h_ref, agg_ref, eps_ref, w1_ref, b1_ref, w2_ref, b2_ref,
              out_ref, *, mid_relu):
    pre = h_ref[...] * eps_ref[...] + agg_ref[0] + agg_ref[1]
    z = lax.dot(pre, w1_ref[...], precision=lax.Precision.HIGHEST) + b1_ref[...]
    z = jnp.maximum(z, 0.0)
    h2 = lax.dot(z, w2_ref[...], precision=lax.Precision.HIGHEST) + b2_ref[...]
    out_ref[...] = jnp.maximum(h2, 0.0) if mid_relu else h2


def _mlp(h, agg, epsb, w1f, b1f, w2f, b2f, mid_relu):
    B = 2000
    return pl.pallas_call(
        functools.partial(_mlp_body, mid_relu=mid_relu),
        grid=(_N // B,),
        in_specs=[pl.BlockSpec((B, _D), lambda i: (i, 0)),
                  pl.BlockSpec((2, B, _D), lambda i: (0, i, 0)),
                  pl.BlockSpec((1, _D), lambda i: (0, 0)),
                  pl.BlockSpec((_D, 2 * _D), lambda i: (0, 0)),
                  pl.BlockSpec((1, 2 * _D), lambda i: (0, 0)),
                  pl.BlockSpec((2 * _D, _D), lambda i: (0, 0)),
                  pl.BlockSpec((1, _D), lambda i: (0, 0))],
        out_specs=pl.BlockSpec((B, _D), lambda i: (i, 0)),
        out_shape=jax.ShapeDtypeStruct((_N, _D), jnp.float32),
    )(h, agg, epsb, w1f, b1f, w2f, b2f)


def _sc_msg_body(h_hbm, src_hbm, dst_hbm, combo_hbm, ctab_hbm, zeros_hbm,
                 out_hbm, ctab_sh, srcv, dstv, combov, rows_v, crows_v,
                 agg_sh, sem, sem2):
    cid = lax.axis_index("c")
    sid = lax.axis_index("s")
    w = cid * _NS + sid

    @pl.when(sid == 0)
    def _stage_ctab():
        pltpu.sync_copy(ctab_hbm, ctab_sh)

    pltpu.sync_copy(zeros_hbm, agg_sh.at[pl.ds(sid * _RPS, _RPS)])
    plsc.subcore_barrier()

    def chunk_body(g, carry):
        base = w * _EPW + g * _CH
        pltpu.sync_copy(src_hbm.at[pl.ds(base, _CH)], srcv)
        pltpu.sync_copy(dst_hbm.at[pl.ds(base, _CH)], dstv)
        pltpu.sync_copy(combo_hbm.at[pl.ds(base, _CH)], combov)
        cp1 = pltpu.async_copy(h_hbm.at[srcv], rows_v, sem)
        cp2 = pltpu.async_copy(ctab_sh.at[combov], crows_v, sem2)
        cp1.wait()
        cp2.wait()
        for e in range(_CH):
            for j in range(8):
                sl = pl.ds(16 * j, 16)
                rows_v[e, sl] = jnp.maximum(rows_v[e, sl] + crows_v[e, sl],
                                            0.0)
        pltpu.sync_copy(rows_v, agg_sh.at[dstv], add=True)
        return carry

    lax.fori_loop(0, _NCHUNK, chunk_body, 0)
    plsc.subcore_barrier()
    pltpu.sync_copy(agg_sh.at[pl.ds(sid * _RPS, _RPS)],
                    out_hbm.at[cid, pl.ds(sid * _RPS, _RPS)])


def _make_sc_msg():
    mesh = plsc.VectorSubcoreMesh(core_axis_name="c", subcore_axis_name="s")
    return pl.kernel(
        _sc_msg_body,
        mesh=mesh,
        out_type=jax.ShapeDtypeStruct((_NC, _NPAD, _D), jnp.float32),
        scratch_types=[
            pltpu.VMEM_SHARED((60, _D), jnp.float32),
            pltpu.VMEM((_CH,), jnp.int32),
            pltpu.VMEM((_CH,), jnp.int32),
            pltpu.VMEM((_CH,), jnp.int32),
            pltpu.VMEM((_CH, _D), jnp.float32),
            pltpu.VMEM((_CH, _D), jnp.float32),
            pltpu.VMEM_SHARED((_NPAD, _D), jnp.float32),
            pltpu.SemaphoreType.DMA,
            pltpu.SemaphoreType.DMA,
        ],
    )


def kernel(x, edge_index, edge_attr, atom_table, bond_tables, W1, b1, g1,
           be1, W2, b2, bn_g, bn_b, eps_param):
    src = edge_index[0]
    dst = edge_index[1]
    h = _embed_h0(x, atom_table)
    combo = _combo_idx(edge_attr)

    ctabs = jnp.einsum("cf,lfd->lcd", jnp.asarray(_ONEHOT60), bond_tables)
    bn_inv = 1.0 / jnp.sqrt(1.0 + 1e-5)
    c1 = bn_inv * g1
    w1f = W1 * c1[:, None, :]
    b1f = b1 * c1 + be1
    c2 = bn_inv * bn_g
    w2f = W2 * c2[:, None, :]
    b2f = b2 * c2 + bn_b

    zeros = jnp.zeros((_RPS, _D), jnp.float32)
    sc_msg = _make_sc_msg()
    ones_row = jnp.ones((1, _D), jnp.float32)
    for l in range(_L):
        agg = sc_msg(h, src, dst, combo, ctabs[l], zeros)
        h = _mlp(h, agg, (1.0 + eps_param[l]) * ones_row,
                 w1f[l], b1f[l][None, :], w2f[l], b2f[l][None, :],
                 mid_relu=(l < _L - 1))
    return h


# speedup vs baseline: 7.7567x; 2.1834x over previous
import functools

import numpy as np
import jax
import jax.numpy as jnp
from jax import lax
from jax.experimental import pallas as pl
from jax.experimental.pallas import tpu as pltpu
from jax.experimental.pallas import tpu_sc as plsc

_ATOM_DIMS = (119, 4, 12, 12, 10, 6, 6, 2, 2)
_BOND_DIMS = (5, 6, 2)
_N, _E, _D, _L = 10000, 320000, 128, 6
_AF = sum(_ATOM_DIMS)
_BF = sum(_BOND_DIMS)

_NC, _NS = 2, 16
_NW = _NC * _NS
_EPW = _E // _NW
_CH = 80
_NCHUNK = _EPW // _CH
_NPAD = 10240
_RPS = _NPAD // _NS

_oh = np.zeros((60, _BF), np.float32)
for _c in range(60):
    _a0, _r = divmod(_c, 12)
    _a1, _a2 = divmod(_r, 2)
    _oh[_c, _a0] = 1.0
    _oh[_c, 5 + _a1] = 1.0
    _oh[_c, 11 + _a2] = 1.0
_ONEHOT60 = _oh


def _first_argmax_onehot(v, d):
    m = jnp.max(v, axis=1, keepdims=True)
    iota = lax.broadcasted_iota(jnp.int32, v.shape, 1)
    idx = jnp.min(jnp.where(v == m, iota, d), axis=1, keepdims=True)
    return iota == idx, idx


def _h0_body(x_ref, tab_ref, out_ref):
    xv = x_ref[...]
    cols = []
    start = 0
    for d in _ATOM_DIMS:
        onehot, _ = _first_argmax_onehot(xv[:, start:start + d], d)
        cols.append(onehot.astype(jnp.float32))
        start += d
    onehot = jnp.concatenate(cols, axis=1)
    out_ref[...] = lax.dot(onehot, tab_ref[...],
                           precision=lax.Precision.HIGHEST)


def _embed_h0(x, atom_table):
    B = 2000
    return pl.pallas_call(
        _h0_body,
        grid=(_N // B,),
        in_specs=[pl.BlockSpec((B, _AF), lambda i: (i, 0)),
                  pl.BlockSpec((_AF, _D), lambda i: (0, 0))],
        out_specs=pl.BlockSpec((B, _D), lambda i: (i, 0)),
        out_shape=jax.ShapeDtypeStruct((_N, _D), jnp.float32),
    )(x, atom_table)


def _combo_body(ea_ref, out_ref, *, block):
    v = ea_ref[...]
    idxs = []
    start = 0
    for d in _BOND_DIMS:
        _, idx = _first_argmax_onehot(v[:, start:start + d], d)
        idxs.append(idx[:, 0])
        start += d
    i = pl.program_id(0)
    out_ref[pl.ds(i * block, block)] = idxs[0] * 12 + idxs[1] * 2 + idxs[2]


def _combo_idx(edge_attr):
    B = 6400
    return pl.pallas_call(
        functools.partial(_combo_body, block=B),
        grid=(_E // B,),
        in_specs=[pl.BlockSpec((B, _BF), lambda i: (i, 0))],
        out_specs=pl.BlockSpec((_E,), lambda i: (0,)),
        out_shape=jax.ShapeDtypeStruct((_E,), jnp.int32),
    )(edge_attr)


def _mlp_body(h_ref, agg_ref, eps_ref, w1_ref, b1_ref, w2_ref, b2_ref,
              out_ref, *, mid_relu):
    pre = h_ref[...] * eps_ref[...] + agg_ref[0] + agg_ref[1]
    z = lax.dot(pre, w1_ref[...], precision=lax.Precision.HIGHEST) + b1_ref[...]
    z = jnp.maximum(z, 0.0)
    h2 = lax.dot(z, w2_ref[...], precision=lax.Precision.HIGHEST) + b2_ref[...]
    out_ref[...] = jnp.maximum(h2, 0.0) if mid_relu else h2


def _mlp(h, agg, epsb, w1f, b1f, w2f, b2f, mid_relu):
    B = 2000
    return pl.pallas_call(
        functools.partial(_mlp_body, mid_relu=mid_relu),
        grid=(_N // B,),
        in_specs=[pl.BlockSpec((B, _D), lambda i: (i, 0)),
                  pl.BlockSpec((2, B, _D), lambda i: (0, i, 0)),
                  pl.BlockSpec((1, _D), lambda i: (0, 0)),
                  pl.BlockSpec((_D, 2 * _D), lambda i: (0, 0)),
                  pl.BlockSpec((1, 2 * _D), lambda i: (0, 0)),
                  pl.BlockSpec((2 * _D, _D), lambda i: (0, 0)),
                  pl.BlockSpec((1, _D), lambda i: (0, 0))],
        out_specs=pl.BlockSpec((B, _D), lambda i: (i, 0)),
        out_shape=jax.ShapeDtypeStruct((_N, _D), jnp.float32),
    )(h, agg, epsb, w1f, b1f, w2f, b2f)


def _sc_msg_body(h_hbm, epk_hbm, ctab_hbm, zeros_hbm,
                 out_hbm, ctab_sh, idx3, rows, crows, agg_sh, semg, semc):
    cid = lax.axis_index("c")
    sid = lax.axis_index("s")
    w = cid * _NS + sid
    q0 = w * _NCHUNK

    @pl.when(sid == 0)
    def _stage_ctab():
        pltpu.sync_copy(ctab_hbm, ctab_sh)

    pltpu.sync_copy(zeros_hbm, agg_sh.at[pl.ds(sid * _RPS, _RPS)])
    plsc.subcore_barrier()

    def fetch(q, slot):
        pltpu.sync_copy(epk_hbm.at[q], idx3.at[slot])
        pltpu.async_copy(h_hbm.at[idx3.at[slot, 0]], rows.at[slot], semg)
        pltpu.async_copy(ctab_sh.at[idx3.at[slot, 2]], crows.at[slot], semc)

    def process(slot):
        pltpu.make_async_copy(h_hbm.at[idx3.at[slot, 0]], rows.at[slot],
                              semg).wait()
        pltpu.make_async_copy(ctab_sh.at[idx3.at[slot, 2]], crows.at[slot],
                              semc).wait()
        def ebody(e, carry):
            for j in range(8):
                sl = pl.ds(16 * j, 16)
                rows[slot, e, sl] = jnp.maximum(
                    rows[slot, e, sl] + crows[slot, e, sl], 0.0)
            return carry

        lax.fori_loop(0, _CH, ebody, 0)
        pltpu.sync_copy(rows.at[slot], agg_sh.at[idx3.at[slot, 1]], add=True)

    fetch(q0, 0)

    def body2(k, carry):
        g0 = q0 + 2 * k
        fetch(g0 + 1, 1)
        process(0)
        fetch(g0 + 2, 0)
        process(1)
        return carry

    lax.fori_loop(0, (_NCHUNK - 1) // 2, body2, 0)
    process(0)
    plsc.subcore_barrier()
    pltpu.sync_copy(agg_sh.at[pl.ds(sid * _RPS, _RPS)],
                    out_hbm.at[cid, pl.ds(sid * _RPS, _RPS)])


def _make_sc_msg():
    mesh = plsc.VectorSubcoreMesh(core_axis_name="c", subcore_axis_name="s")
    return pl.kernel(
        _sc_msg_body,
        mesh=mesh,
        out_type=jax.ShapeDtypeStruct((_NC, _NPAD, _D), jnp.float32),
        scratch_types=[
            pltpu.VMEM_SHARED((60, _D), jnp.float32),
            pltpu.VMEM((2, 3, _CH), jnp.int32),
            pltpu.VMEM((2, _CH, _D), jnp.float32),
            pltpu.VMEM((2, _CH, _D), jnp.float32),
            pltpu.VMEM_SHARED((_NPAD, _D), jnp.float32),
            pltpu.SemaphoreType.DMA,
            pltpu.SemaphoreType.DMA,
        ],
    )


def kernel(x, edge_index, edge_attr, atom_table, bond_tables, W1, b1, g1,
           be1, W2, b2, bn_g, bn_b, eps_param):
    src = edge_index[0]
    dst = edge_index[1]
    h = _embed_h0(x, atom_table)
    combo = _combo_idx(edge_attr)

    ctabs = jnp.einsum("cf,lfd->lcd", jnp.asarray(_ONEHOT60), bond_tables)
    bn_inv = 1.0 / jnp.sqrt(1.0 + 1e-5)
    c1 = bn_inv * g1
    w1f = W1 * c1[:, None, :]
    b1f = b1 * c1 + be1
    c2 = bn_inv * bn_g
    w2f = W2 * c2[:, None, :]
    b2f = b2 * c2 + bn_b

    zeros = jnp.zeros((_RPS, _D), jnp.float32)
    epk = (jnp.stack([src, dst, combo])
           .reshape(3, _NW, _NCHUNK, _CH)
           .transpose(1, 2, 0, 3)
           .reshape(_NW * _NCHUNK, 3, _CH))
    sc_msg = _make_sc_msg()
    ones_row = jnp.ones((1, _D), jnp.float32)
    for l in range(_L):
        agg = sc_msg(h, epk, ctabs[l], zeros)
        h = _mlp(h, agg, (1.0 + eps_param[l]) * ones_row,
                 w1f[l], b1f[l][None, :], w2f[l], b2f[l][None, :],
                 mid_relu=(l < _L - 1))
    return h


# speedup vs baseline: 8.1117x; 1.0458x over previous
import functools

import numpy as np
import jax
import jax.numpy as jnp
from jax import lax
from jax.experimental import pallas as pl
from jax.experimental.pallas import tpu as pltpu
from jax.experimental.pallas import tpu_sc as plsc

_ATOM_DIMS = (119, 4, 12, 12, 10, 6, 6, 2, 2)
_BOND_DIMS = (5, 6, 2)
_N, _E, _D, _L = 10000, 320000, 128, 6
_AF = sum(_ATOM_DIMS)
_BF = sum(_BOND_DIMS)

_NC, _NS = 2, 16
_NW = _NC * _NS
_EPW = _E // _NW
_CH = 40
_NCHUNK = _EPW // _CH
_BLK = 10
_NBLK = _NCHUNK // _BLK
_NPAD = 10112
_RPS = _NPAD // _NS

_oh = np.zeros((60, _BF), np.float32)
for _c in range(60):
    _a0, _r = divmod(_c, 12)
    _a1, _a2 = divmod(_r, 2)
    _oh[_c, _a0] = 1.0
    _oh[_c, 5 + _a1] = 1.0
    _oh[_c, 11 + _a2] = 1.0
_ONEHOT60 = _oh


def _first_argmax_onehot(v, d):
    m = jnp.max(v, axis=1, keepdims=True)
    iota = lax.broadcasted_iota(jnp.int32, v.shape, 1)
    idx = jnp.min(jnp.where(v == m, iota, d), axis=1, keepdims=True)
    return iota == idx, idx


def _h0_body(x_ref, tab_ref, out_ref):
    xv = x_ref[...]
    cols = []
    start = 0
    for d in _ATOM_DIMS:
        onehot, _ = _first_argmax_onehot(xv[:, start:start + d], d)
        cols.append(onehot.astype(jnp.float32))
        start += d
    onehot = jnp.concatenate(cols, axis=1)
    out_ref[...] = lax.dot(onehot, tab_ref[...],
                           precision=lax.Precision.HIGHEST)


def _embed_h0(x, atom_table):
    B = 2000
    return pl.pallas_call(
        _h0_body,
        grid=(_N // B,),
        in_specs=[pl.BlockSpec((B, _AF), lambda i: (i, 0)),
                  pl.BlockSpec((_AF, _D), lambda i: (0, 0))],
        out_specs=pl.BlockSpec((B, _D), lambda i: (i, 0)),
        out_shape=jax.ShapeDtypeStruct((_N, _D), jnp.float32),
    )(x, atom_table)


def _combo_body(ea_ref, out_ref, *, block):
    v = ea_ref[...]
    idxs = []
    start = 0
    for d in _BOND_DIMS:
        _, idx = _first_argmax_onehot(v[:, start:start + d], d)
        idxs.append(idx[:, 0])
        start += d
    i = pl.program_id(0)
    out_ref[pl.ds(i * block, block)] = idxs[0] * 12 + idxs[1] * 2 + idxs[2]


def _combo_idx(edge_attr):
    B = 6400
    return pl.pallas_call(
        functools.partial(_combo_body, block=B),
        grid=(_E // B,),
        in_specs=[pl.BlockSpec((B, _BF), lambda i: (i, 0))],
        out_specs=pl.BlockSpec((_E,), lambda i: (0,)),
        out_shape=jax.ShapeDtypeStruct((_E,), jnp.int32),
    )(edge_attr)


def _mlp_body(h_ref, agg_ref, eps_ref, w1_ref, b1_ref, w2_ref, b2_ref,
              out_ref, *, mid_relu):
    pre = h_ref[...] * eps_ref[...] + agg_ref[0] + agg_ref[1]
    z = lax.dot(pre, w1_ref[...], precision=lax.Precision.HIGHEST) + b1_ref[...]
    z = jnp.maximum(z, 0.0)
    h2 = lax.dot(z, w2_ref[...], precision=lax.Precision.HIGHEST) + b2_ref[...]
    out_ref[...] = jnp.maximum(h2, 0.0) if mid_relu else h2


def _mlp(h, agg, epsb, w1f, b1f, w2f, b2f, mid_relu):
    B = 2000
    return pl.pallas_call(
        functools.partial(_mlp_body, mid_relu=mid_relu),
        grid=(_N // B,),
        in_specs=[pl.BlockSpec((B, _D), lambda i: (i, 0)),
                  pl.BlockSpec((2, B, _D), lambda i: (0, i, 0)),
                  pl.BlockSpec((1, _D), lambda i: (0, 0)),
                  pl.BlockSpec((_D, 2 * _D), lambda i: (0, 0)),
                  pl.BlockSpec((1, 2 * _D), lambda i: (0, 0)),
                  pl.BlockSpec((2 * _D, _D), lambda i: (0, 0)),
                  pl.BlockSpec((1, _D), lambda i: (0, 0))],
        out_specs=pl.BlockSpec((B, _D), lambda i: (i, 0)),
        out_shape=jax.ShapeDtypeStruct((_N, _D), jnp.float32),
    )(h, agg, epsb, w1f, b1f, w2f, b2f)


def _sc_msg_body(h_hbm, epk_hbm, ctab_hbm, zeros_hbm, out_hbm,
                 ctab_sh, idxblk, rows, crows, agg_sh,
                 semi, semg0, semg1, semg2, semc0, semc1,
                 sems0, sems1, sems2):
    cid = lax.axis_index("c")
    sid = lax.axis_index("s")
    w = cid * _NS + sid
    q0 = w * _NCHUNK
    semg = (semg0, semg1, semg2)
    semc = (semc0, semc1)
    sems = (sems0, sems1, sems2)

    @pl.when(sid == 0)
    def _stage_ctab():
        pltpu.sync_copy(ctab_hbm, ctab_sh)

    pltpu.sync_copy(zeros_hbm, agg_sh.at[pl.ds(sid * _RPS, _RPS)])
    plsc.subcore_barrier()

    def gathers(bs, c, rs, cs):
        pltpu.async_copy(h_hbm.at[idxblk.at[bs, c, 0]], rows.at[rs],
                         semg[rs])
        pltpu.async_copy(ctab_sh.at[idxblk.at[bs, c, 2]], crows.at[cs],
                         semc[cs])

    def wait_sc(rs):
        pltpu.make_async_copy(rows.at[rs], agg_sh.at[idxblk.at[0, 0, 1]],
                              sems[rs]).wait()

    def process(bs, c, rs, cs):
        pltpu.make_async_copy(h_hbm.at[idxblk.at[bs, c, 0]], rows.at[rs],
                              semg[rs]).wait()
        pltpu.make_async_copy(ctab_sh.at[idxblk.at[bs, c, 2]],
                              crows.at[cs], semc[cs]).wait()

        def ebody(e, carry):
            for j in range(8):
                sl = pl.ds(16 * j, 16)
                rows[rs, e, sl] = jnp.maximum(
                    rows[rs, e, sl] + crows[cs, e, sl], 0.0)
            return carry

        lax.fori_loop(0, _CH, ebody, 0)
        pltpu.async_copy(rows.at[rs], agg_sh.at[idxblk.at[bs, c, 1]],
                         sems[rs], add=True)

    def fetch_blk(q, bs):
        pltpu.async_copy(epk_hbm.at[pl.ds(q, _BLK)], idxblk.at[bs], semi)

    def wait_blk(bs):
        pltpu.make_async_copy(epk_hbm.at[pl.ds(0, _BLK)], idxblk.at[bs],
                              semi).wait()

    def run_block(bs, qnext):
        fetch_blk(qnext, 1 - bs)
        gathers(bs, 0, 0, 0)
        gathers(bs, 1, 1, 1)
        for c in range(_BLK):
            process(bs, c, c % 3, c % 2)
            if c + 2 < _BLK:
                if c >= 1:
                    wait_sc((c + 2) % 3)
                gathers(bs, c + 2, (c + 2) % 3, (c + 2) % 2)
        wait_sc((_BLK - 3) % 3)
        wait_sc((_BLK - 2) % 3)
        wait_sc((_BLK - 1) % 3)

    fetch_blk(q0, 0)
    wait_blk(0)

    def bodyblk(kb, carry):
        qb = q0 + 2 * kb * _BLK
        run_block(0, qb + _BLK)
        wait_blk(1)
        run_block(1, qb + 2 * _BLK)
        wait_blk(0)
        return carry

    lax.fori_loop(0, _NBLK // 2, bodyblk, 0)
    run_block(0, q0 + _NCHUNK)
    wait_blk(1)
    plsc.subcore_barrier()
    pltpu.sync_copy(agg_sh.at[pl.ds(sid * _RPS, _RPS)],
                    out_hbm.at[cid, pl.ds(sid * _RPS, _RPS)])


def _make_sc_msg():
    mesh = plsc.VectorSubcoreMesh(core_axis_name="c", subcore_axis_name="s")
    return pl.kernel(
        _sc_msg_body,
        mesh=mesh,
        out_type=jax.ShapeDtypeStruct((_NC, _NPAD, _D), jnp.float32),
        scratch_types=[
            pltpu.VMEM_SHARED((60, _D), jnp.float32),
            pltpu.VMEM((2, _BLK, 3, _CH), jnp.int32),
            pltpu.VMEM((3, _CH, _D), jnp.float32),
            pltpu.VMEM((2, _CH, _D), jnp.float32),
            pltpu.VMEM_SHARED((_NPAD, _D), jnp.float32),
        ] + [pltpu.SemaphoreType.DMA] * 9,
    )


def kernel(x, edge_index, edge_attr, atom_table, bond_tables, W1, b1, g1,
           be1, W2, b2, bn_g, bn_b, eps_param):
    src = edge_index[0]
    dst = edge_index[1]
    h = _embed_h0(x, atom_table)
    combo = _combo_idx(edge_attr)

    ctabs = jnp.einsum("cf,lfd->lcd", jnp.asarray(_ONEHOT60), bond_tables)
    bn_inv = 1.0 / jnp.sqrt(1.0 + 1e-5)
    c1 = bn_inv * g1
    w1f = W1 * c1[:, None, :]
    b1f = b1 * c1 + be1
    c2 = bn_inv * bn_g
    w2f = W2 * c2[:, None, :]
    b2f = b2 * c2 + bn_b

    zeros = jnp.zeros((_RPS, _D), jnp.float32)
    epk = (jnp.stack([src, dst, combo])
           .reshape(3, _NW, _NCHUNK, _CH)
           .transpose(1, 2, 0, 3)
           .reshape(_NW * _NCHUNK, 3, _CH))
    epk = jnp.pad(epk, ((0, _BLK), (0, 0), (0, 0)))
    sc_msg = _make_sc_msg()
    ones_row = jnp.ones((1, _D), jnp.float32)
    for l in range(_L):
        agg = sc_msg(h, epk, ctabs[l], zeros)
        h = _mlp(h, agg, (1.0 + eps_param[l]) * ones_row,
                 w1f[l], b1f[l][None, :], w2f[l], b2f[l][None, :],
                 mid_relu=(l < _L - 1))
    return h


# speedup vs baseline: 8.6096x; 1.0614x over previous
import functools

import numpy as np
import jax
import jax.numpy as jnp
from jax import lax
from jax.experimental import pallas as pl
from jax.experimental.pallas import tpu as pltpu
from jax.experimental.pallas import tpu_sc as plsc

_ATOM_DIMS = (119, 4, 12, 12, 10, 6, 6, 2, 2)
_BOND_DIMS = (5, 6, 2)
_N, _E, _D, _L = 10000, 320000, 128, 6
_AF = sum(_ATOM_DIMS)
_BF = sum(_BOND_DIMS)

_NC, _NS = 2, 16
_NW = _NC * _NS
_EPW = _E // _NW
_CH = 40
_NCHUNK = _EPW // _CH
_BLK = 10
_NBLK = _NCHUNK // _BLK
_NPAD = 10112
_RPS = _NPAD // _NS

_oh = np.zeros((60, _BF), np.float32)
for _c in range(60):
    _a0, _r = divmod(_c, 12)
    _a1, _a2 = divmod(_r, 2)
    _oh[_c, _a0] = 1.0
    _oh[_c, 5 + _a1] = 1.0
    _oh[_c, 11 + _a2] = 1.0
_ONEHOT60 = _oh


def _first_argmax_onehot(v, d):
    m = jnp.max(v, axis=1, keepdims=True)
    iota = lax.broadcasted_iota(jnp.int32, v.shape, 1)
    idx = jnp.min(jnp.where(v == m, iota, d), axis=1, keepdims=True)
    return iota == idx, idx


def _h0_body(x_ref, tab_ref, out_ref):
    xv = x_ref[...]
    cols = []
    start = 0
    for d in _ATOM_DIMS:
        onehot, _ = _first_argmax_onehot(xv[:, start:start + d], d)
        cols.append(onehot.astype(jnp.float32))
        start += d
    onehot = jnp.concatenate(cols, axis=1)
    out_ref[...] = lax.dot(onehot, tab_ref[...],
                           precision=lax.Precision.HIGHEST)


def _embed_h0(x, atom_table):
    B = 2000
    return pl.pallas_call(
        _h0_body,
        grid=(_N // B,),
        in_specs=[pl.BlockSpec((B, _AF), lambda i: (i, 0)),
                  pl.BlockSpec((_AF, _D), lambda i: (0, 0))],
        out_specs=pl.BlockSpec((B, _D), lambda i: (i, 0)),
        out_shape=jax.ShapeDtypeStruct((_N, _D), jnp.float32),
    )(x, atom_table)


def _combo_body(ea_ref, out_ref, *, block):
    v = ea_ref[...]
    idxs = []
    start = 0
    for d in _BOND_DIMS:
        _, idx = _first_argmax_onehot(v[:, start:start + d], d)
        idxs.append(idx[:, 0])
        start += d
    i = pl.program_id(0)
    out_ref[pl.ds(i * block, block)] = idxs[0] * 12 + idxs[1] * 2 + idxs[2]


def _combo_idx(edge_attr):
    B = 6400
    return pl.pallas_call(
        functools.partial(_combo_body, block=B),
        grid=(_E // B,),
        in_specs=[pl.BlockSpec((B, _BF), lambda i: (i, 0))],
        out_specs=pl.BlockSpec((_E,), lambda i: (0,)),
        out_shape=jax.ShapeDtypeStruct((_E,), jnp.int32),
    )(edge_attr)


def _mlp_body(h_ref, agg_ref, eps_ref, w1_ref, b1_ref, w2_ref, b2_ref,
              out_ref, *, mid_relu):
    pre = h_ref[...] * eps_ref[...] + agg_ref[0] + agg_ref[1]
    z = lax.dot(pre, w1_ref[...]) + b1_ref[...]
    z = jnp.maximum(z, 0.0)
    h2 = lax.dot(z, w2_ref[...]) + b2_ref[...]
    out_ref[...] = jnp.maximum(h2, 0.0) if mid_relu else h2


def _mlp(h, agg, epsb, w1f, b1f, w2f, b2f, mid_relu):
    B = 2000
    return pl.pallas_call(
        functools.partial(_mlp_body, mid_relu=mid_relu),
        grid=(_N // B,),
        in_specs=[pl.BlockSpec((B, _D), lambda i: (i, 0)),
                  pl.BlockSpec((2, B, _D), lambda i: (0, i, 0)),
                  pl.BlockSpec((1, _D), lambda i: (0, 0)),
                  pl.BlockSpec((_D, 2 * _D), lambda i: (0, 0)),
                  pl.BlockSpec((1, 2 * _D), lambda i: (0, 0)),
                  pl.BlockSpec((2 * _D, _D), lambda i: (0, 0)),
                  pl.BlockSpec((1, _D), lambda i: (0, 0))],
        out_specs=pl.BlockSpec((B, _D), lambda i: (i, 0)),
        out_shape=jax.ShapeDtypeStruct((_N, _D), jnp.float32),
    )(h, agg, epsb, w1f, b1f, w2f, b2f)


def _sc_msg_body(h_hbm, epk_hbm, ctab_hbm, zeros_hbm, out_hbm,
                 ctab_sh, idxblk, rows, crows, agg_sh,
                 semi, semg0, semg1, semg2, semc0, semc1,
                 sems0, sems1, sems2):
    cid = lax.axis_index("c")
    sid = lax.axis_index("s")
    w = cid * _NS + sid
    q0 = w * _NCHUNK
    semg = (semg0, semg1, semg2)
    semc = (semc0, semc1)
    sems = (sems0, sems1, sems2)

    @pl.when(sid == 0)
    def _stage_ctab():
        pltpu.sync_copy(ctab_hbm, ctab_sh)

    pltpu.sync_copy(zeros_hbm, agg_sh.at[pl.ds(sid * _RPS, _RPS)])
    plsc.subcore_barrier()

    def gathers(bs, c, rs, cs):
        pltpu.async_copy(h_hbm.at[idxblk.at[bs, c, 0]], rows.at[rs],
                         semg[rs])
        pltpu.async_copy(ctab_sh.at[idxblk.at[bs, c, 2]], crows.at[cs],
                         semc[cs])

    def wait_sc(rs):
        pltpu.make_async_copy(rows.at[rs], agg_sh.at[idxblk.at[0, 0, 1]],
                              sems[rs]).wait()

    def process(bs, c, rs, cs):
        pltpu.make_async_copy(h_hbm.at[idxblk.at[bs, c, 0]], rows.at[rs],
                              semg[rs]).wait()
        pltpu.make_async_copy(ctab_sh.at[idxblk.at[bs, c, 2]],
                              crows.at[cs], semc[cs]).wait()

        def ebody(e2, carry):
            for de in range(2):
                e = 2 * e2 + de
                for j in range(8):
                    sl = pl.ds(16 * j, 16)
                    rows[rs, e, sl] = jnp.maximum(
                        rows[rs, e, sl] + crows[cs, e, sl], 0.0)
            return carry

        lax.fori_loop(0, _CH // 2, ebody, 0)
        pltpu.async_copy(rows.at[rs], agg_sh.at[idxblk.at[bs, c, 1]],
                         sems[rs], add=True)

    def fetch_blk(q, bs):
        pltpu.async_copy(epk_hbm.at[pl.ds(q, _BLK)], idxblk.at[bs], semi)

    def wait_blk(bs):
        pltpu.make_async_copy(epk_hbm.at[pl.ds(0, _BLK)], idxblk.at[bs],
                              semi).wait()

    def run_block(bs, qnext):
        fetch_blk(qnext, 1 - bs)
        gathers(bs, 0, 0, 0)
        gathers(bs, 1, 1, 1)
        for c in range(_BLK):
            process(bs, c, c % 3, c % 2)
            if c + 2 < _BLK:
                if c >= 1:
                    wait_sc((c + 2) % 3)
                gathers(bs, c + 2, (c + 2) % 3, (c + 2) % 2)
        wait_sc((_BLK - 3) % 3)
        wait_sc((_BLK - 2) % 3)
        wait_sc((_BLK - 1) % 3)

    fetch_blk(q0, 0)
    wait_blk(0)

    def bodyblk(kb, carry):
        qb = q0 + 2 * kb * _BLK
        run_block(0, qb + _BLK)
        wait_blk(1)
        run_block(1, qb + 2 * _BLK)
        wait_blk(0)
        return carry

    lax.fori_loop(0, _NBLK // 2, bodyblk, 0)
    run_block(0, q0 + _NCHUNK)
    wait_blk(1)
    plsc.subcore_barrier()
    pltpu.sync_copy(agg_sh.at[pl.ds(sid * _RPS, _RPS)],
                    out_hbm.at[cid, pl.ds(sid * _RPS, _RPS)])


def _make_sc_msg():
    mesh = plsc.VectorSubcoreMesh(core_axis_name="c", subcore_axis_name="s")
    return pl.kernel(
        _sc_msg_body,
        mesh=mesh,
        out_type=jax.ShapeDtypeStruct((_NC, _NPAD, _D), jnp.float32),
        scratch_types=[
            pltpu.VMEM_SHARED((60, _D), jnp.float32),
            pltpu.VMEM((2, _BLK, 3, _CH), jnp.int32),
            pltpu.VMEM((3, _CH, _D), jnp.float32),
            pltpu.VMEM((2, _CH, _D), jnp.float32),
            pltpu.VMEM_SHARED((_NPAD, _D), jnp.float32),
        ] + [pltpu.SemaphoreType.DMA] * 9,
    )


def kernel(x, edge_index, edge_attr, atom_table, bond_tables, W1, b1, g1,
           be1, W2, b2, bn_g, bn_b, eps_param):
    src = edge_index[0]
    dst = edge_index[1]
    h = _embed_h0(x, atom_table)
    combo = _combo_idx(edge_attr)

    ctabs = jnp.einsum("cf,lfd->lcd", jnp.asarray(_ONEHOT60), bond_tables)
    bn_inv = 1.0 / jnp.sqrt(1.0 + 1e-5)
    c1 = bn_inv * g1
    w1f = W1 * c1[:, None, :]
    b1f = b1 * c1 + be1
    c2 = bn_inv * bn_g
    w2f = W2 * c2[:, None, :]
    b2f = b2 * c2 + bn_b

    zeros = jnp.zeros((_RPS, _D), jnp.float32)
    epk = (jnp.stack([src, dst, combo])
           .reshape(3, _NW, _NCHUNK, _CH)
           .transpose(1, 2, 0, 3)
           .reshape(_NW * _NCHUNK, 3, _CH))
    epk = jnp.pad(epk, ((0, _BLK), (0, 0), (0, 0)))
    sc_msg = _make_sc_msg()
    ones_row = jnp.ones((1, _D), jnp.float32)
    for l in range(_L):
        agg = sc_msg(h, epk, ctabs[l], zeros)
        h = _mlp(h, agg, (1.0 + eps_param[l]) * ones_row,
                 w1f[l], b1f[l][None, :], w2f[l], b2f[l][None, :],
                 mid_relu=(l < _L - 1))
    return h


# speedup vs baseline: 9.4381x; 1.0962x over previous
import functools

import numpy as np
import jax
import jax.numpy as jnp
from jax import lax
from jax.experimental import pallas as pl
from jax.experimental.pallas import tpu as pltpu
from jax.experimental.pallas import tpu_sc as plsc

_ATOM_DIMS = (119, 4, 12, 12, 10, 6, 6, 2, 2)
_BOND_DIMS = (5, 6, 2)
_N, _E, _D, _L = 10000, 320000, 128, 6
_AF = sum(_ATOM_DIMS)
_BF = sum(_BOND_DIMS)

_NC, _NS = 2, 16
_NW = _NC * _NS
_EPW = _E // _NW
_CH = 40
_NCHUNK = _EPW // _CH
_BLK = 10
_NBLK = _NCHUNK // _BLK
_NPAD = 10112
_RPS = _NPAD // _NS

_oh = np.zeros((60, _BF), np.float32)
for _c in range(60):
    _a0, _r = divmod(_c, 12)
    _a1, _a2 = divmod(_r, 2)
    _oh[_c, _a0] = 1.0
    _oh[_c, 5 + _a1] = 1.0
    _oh[_c, 11 + _a2] = 1.0
_ONEHOT60 = _oh


def _first_argmax_onehot(v, d):
    m = jnp.max(v, axis=1, keepdims=True)
    iota = lax.broadcasted_iota(jnp.int32, v.shape, 1)
    idx = jnp.min(jnp.where(v == m, iota, d), axis=1, keepdims=True)
    return iota == idx, idx


def _h0_body(x_ref, tab_ref, out_ref):
    xv = x_ref[...]
    cols = []
    start = 0
    for d in _ATOM_DIMS:
        onehot, _ = _first_argmax_onehot(xv[:, start:start + d], d)
        cols.append(onehot.astype(jnp.float32))
        start += d
    onehot = jnp.concatenate(cols, axis=1)
    out_ref[...] = lax.dot(onehot, tab_ref[...],
                           precision=lax.Precision.HIGHEST)


def _embed_h0(x, atom_table):
    B = 2000
    return pl.pallas_call(
        _h0_body,
        grid=(_N // B,),
        in_specs=[pl.BlockSpec((B, _AF), lambda i: (i, 0)),
                  pl.BlockSpec((_AF, _D), lambda i: (0, 0))],
        out_specs=pl.BlockSpec((B, _D), lambda i: (i, 0)),
        out_shape=jax.ShapeDtypeStruct((_N, _D), jnp.float32),
    )(x, atom_table)


def _combo_body(ea_ref, out_ref, *, block):
    v = ea_ref[...]
    idxs = []
    start = 0
    for d in _BOND_DIMS:
        _, idx = _first_argmax_onehot(v[:, start:start + d], d)
        idxs.append(idx[:, 0])
        start += d
    i = pl.program_id(0)
    out_ref[pl.ds(i * block, block)] = idxs[0] * 12 + idxs[1] * 2 + idxs[2]


def _combo_idx(edge_attr):
    B = 6400
    return pl.pallas_call(
        functools.partial(_combo_body, block=B),
        grid=(_E // B,),
        in_specs=[pl.BlockSpec((B, _BF), lambda i: (i, 0))],
        out_specs=pl.BlockSpec((_E,), lambda i: (0,)),
        out_shape=jax.ShapeDtypeStruct((_E,), jnp.int32),
    )(edge_attr)


def _mlp_body(h_ref, agg_ref, eps_ref, w1_ref, b1_ref, w2_ref, b2_ref,
              out_ref, *, mid_relu):
    pre = h_ref[...] * eps_ref[...] + agg_ref[0] + agg_ref[1]
    z = lax.dot(pre, w1_ref[...]) + b1_ref[...]
    z = jnp.maximum(z, 0.0)
    h2 = lax.dot(z, w2_ref[...]) + b2_ref[...]
    out_ref[...] = jnp.maximum(h2, 0.0) if mid_relu else h2


def _mlp(h, agg, epsb, w1f, b1f, w2f, b2f, mid_relu):
    B = 2000
    return pl.pallas_call(
        functools.partial(_mlp_body, mid_relu=mid_relu),
        grid=(_N // B,),
        in_specs=[pl.BlockSpec((B, _D), lambda i: (i, 0)),
                  pl.BlockSpec((2, B, _D), lambda i: (0, i, 0)),
                  pl.BlockSpec((1, _D), lambda i: (0, 0)),
                  pl.BlockSpec((_D, 2 * _D), lambda i: (0, 0)),
                  pl.BlockSpec((1, 2 * _D), lambda i: (0, 0)),
                  pl.BlockSpec((2 * _D, _D), lambda i: (0, 0)),
                  pl.BlockSpec((1, _D), lambda i: (0, 0))],
        out_specs=pl.BlockSpec((B, _D), lambda i: (i, 0)),
        out_shape=jax.ShapeDtypeStruct((_N, _D), jnp.float32),
    )(h, agg, epsb, w1f, b1f, w2f, b2f)


def _sc_msg_body(h_hbm, epk_hbm, ctab_hbm, zeros_hbm, out_hbm,
                 ctab_sh, idxblk, rows, crows, agg_sh,
                 semi, semg0, semg1, semg2, semc0, semc1,
                 sems0, sems1, sems2):
    cid = lax.axis_index("c")
    sid = lax.axis_index("s")
    w = cid * _NS + sid
    q0 = w * _NCHUNK
    semg = (semg0, semg1, semg2)
    semc = (semc0, semc1)
    sems = (sems0, sems1, sems2)

    @pl.when(sid == 0)
    def _stage_ctab():
        pltpu.sync_copy(ctab_hbm, ctab_sh)

    pltpu.sync_copy(zeros_hbm, agg_sh.at[pl.ds(sid * _RPS, _RPS)])
    plsc.subcore_barrier()

    def gathers(bs, c, rs, cs):
        pltpu.async_copy(h_hbm.at[idxblk.at[bs, c, 0]], rows.at[rs],
                         semg[rs])
        pltpu.async_copy(ctab_sh.at[idxblk.at[bs, c, 2]], crows.at[cs],
                         semc[cs])

    def wait_sc(rs):
        pltpu.make_async_copy(rows.at[rs], agg_sh.at[idxblk.at[0, 0, 1]],
                              sems[rs]).wait()

    def process(bs, c, rs, cs):
        pltpu.make_async_copy(h_hbm.at[idxblk.at[bs, c, 0]], rows.at[rs],
                              semg[rs]).wait()
        pltpu.make_async_copy(ctab_sh.at[idxblk.at[bs, c, 2]],
                              crows.at[cs], semc[cs]).wait()

        @functools.partial(plsc.parallel_loop, 0, _CH, unroll=2)
        def _eloop(e):
            for j in range(8):
                sl = pl.ds(16 * j, 16)
                rows[rs, e, sl] = jnp.maximum(
                    rows[rs, e, sl] + crows[cs, e, sl], 0.0)
        pltpu.async_copy(rows.at[rs], agg_sh.at[idxblk.at[bs, c, 1]],
                         sems[rs], add=True)

    def fetch_blk(q, bs):
        pltpu.async_copy(epk_hbm.at[pl.ds(q, _BLK)], idxblk.at[bs], semi)

    def wait_blk(bs):
        pltpu.make_async_copy(epk_hbm.at[pl.ds(0, _BLK)], idxblk.at[bs],
                              semi).wait()

    def run_block(bs, qnext):
        fetch_blk(qnext, 1 - bs)
        gathers(bs, 0, 0, 0)
        gathers(bs, 1, 1, 1)
        for c in range(_BLK):
            process(bs, c, c % 3, c % 2)
            if c + 2 < _BLK:
                if c >= 1:
                    wait_sc((c + 2) % 3)
                gathers(bs, c + 2, (c + 2) % 3, (c + 2) % 2)
        wait_sc((_BLK - 3) % 3)
        wait_sc((_BLK - 2) % 3)
        wait_sc((_BLK - 1) % 3)

    fetch_blk(q0, 0)
    wait_blk(0)

    def bodyblk(kb, carry):
        qb = q0 + 2 * kb * _BLK
        run_block(0, qb + _BLK)
        wait_blk(1)
        run_block(1, qb + 2 * _BLK)
        wait_blk(0)
        return carry

    lax.fori_loop(0, _NBLK // 2, bodyblk, 0)
    run_block(0, q0 + _NCHUNK)
    wait_blk(1)
    plsc.subcore_barrier()
    pltpu.sync_copy(agg_sh.at[pl.ds(sid * _RPS, _RPS)],
                    out_hbm.at[cid, pl.ds(sid * _RPS, _RPS)])


def _make_sc_msg():
    mesh = plsc.VectorSubcoreMesh(core_axis_name="c", subcore_axis_name="s")
    return pl.kernel(
        _sc_msg_body,
        mesh=mesh,
        out_type=jax.ShapeDtypeStruct((_NC, _NPAD, _D), jnp.float32),
        scratch_types=[
            pltpu.VMEM_SHARED((60, _D), jnp.float32),
            pltpu.VMEM((2, _BLK, 3, _CH), jnp.int32),
            pltpu.VMEM((3, _CH, _D), jnp.float32),
            pltpu.VMEM((2, _CH, _D), jnp.float32),
            pltpu.VMEM_SHARED((_NPAD, _D), jnp.float32),
        ] + [pltpu.SemaphoreType.DMA] * 9,
    )


def kernel(x, edge_index, edge_attr, atom_table, bond_tables, W1, b1, g1,
           be1, W2, b2, bn_g, bn_b, eps_param):
    src = edge_index[0]
    dst = edge_index[1]
    h = _embed_h0(x, atom_table)
    combo = _combo_idx(edge_attr)

    ctabs = jnp.einsum("cf,lfd->lcd", jnp.asarray(_ONEHOT60), bond_tables)
    bn_inv = 1.0 / jnp.sqrt(1.0 + 1e-5)
    c1 = bn_inv * g1
    w1f = W1 * c1[:, None, :]
    b1f = b1 * c1 + be1
    c2 = bn_inv * bn_g
    w2f = W2 * c2[:, None, :]
    b2f = b2 * c2 + bn_b

    zeros = jnp.zeros((_RPS, _D), jnp.float32)
    epk = (jnp.stack([src, dst, combo])
           .reshape(3, _NW, _NCHUNK, _CH)
           .transpose(1, 2, 0, 3)
           .reshape(_NW * _NCHUNK, 3, _CH))
    epk = jnp.pad(epk, ((0, _BLK), (0, 0), (0, 0)))
    sc_msg = _make_sc_msg()
    ones_row = jnp.ones((1, _D), jnp.float32)
    for l in range(_L):
        agg = sc_msg(h, epk, ctabs[l], zeros)
        h = _mlp(h, agg, (1.0 + eps_param[l]) * ones_row,
                 w1f[l], b1f[l][None, :], w2f[l], b2f[l][None, :],
                 mid_relu=(l < _L - 1))
    return h
